# baseline (speedup 1.0000x reference)
import functools

import jax
import jax.numpy as jnp
from jax import lax
from jax.experimental import pallas as pl
from jax.experimental.pallas import tpu as pltpu
from jax.experimental.pallas import tpu_sc as plsc

N_NODES = 10000
D = 128
N_PAD = 10240
NC, NS = 2, 16
NW = NC * NS
CHUNK = 128
ROWS_PER_SUB = N_PAD // NS
ZR = 16
PAD_SRC = N_NODES
PAD_DST = N_NODES + 8
N0C = 104
N1C = 64

_MESH = plsc.VectorSubcoreMesh(core_axis_name="c", subcore_axis_name="s")


def _sc_degree(dst3):
    n_chunks = dst3.shape[1]

    @functools.partial(
        pl.kernel,
        out_type=jax.ShapeDtypeStruct((NC, N_PAD), jnp.float32),
        mesh=_MESH,
        compiler_params=pltpu.CompilerParams(needs_layout_passes=False),
        scratch_types=[
            pltpu.VMEM_SHARED((NS, N_PAD), jnp.float32),
            pltpu.VMEM((n_chunks, CHUNK), jnp.int32),
            pltpu.VMEM((N_PAD,), jnp.float32),
            pltpu.VMEM((NS, ROWS_PER_SUB), jnp.float32),
        ],
    )
    def k(dst3_hbm, degp_hbm, slab, idx_v, hist_v, win_v):
        c = lax.axis_index("c")
        s = lax.axis_index("s")
        wid = c * NS + s

        def fill_zeros(i, _):
            hist_v[pl.ds(i * 16, 16)] = jnp.zeros((16,), jnp.float32)
            return 0

        lax.fori_loop(0, N_PAD // 16, fill_zeros, 0)
        pltpu.sync_copy(dst3_hbm.at[wid], idx_v)
        ones16 = jnp.full((16,), 1.0, jnp.float32)

        def body(j, _):
            def inner(t, _):
                idx = idx_v[j, pl.ds(t * 16, 16)]
                plsc.addupdate_scatter(hist_v, [idx], ones16)
                return 0

            lax.fori_loop(0, CHUNK // 16, inner, 0)
            return 0

        lax.fori_loop(0, n_chunks, body, 0)
        pltpu.sync_copy(hist_v, slab.at[s])
        plsc.subcore_barrier()
        pltpu.sync_copy(slab.at[:, pl.ds(s * ROWS_PER_SUB, ROWS_PER_SUB)],
                        win_v)

        def red(j, _):
            acc = jnp.zeros((16,), jnp.float32)

            def radd(i, a):
                return a + win_v[i, pl.ds(j * 16, 16)]

            acc = lax.fori_loop(0, NS, radd, acc)
            hist_v[pl.ds(j * 16, 16)] = acc
            return 0

        lax.fori_loop(0, ROWS_PER_SUB // 16, red, 0)
        pltpu.sync_copy(hist_v.at[pl.ds(0, ROWS_PER_SUB)],
                        degp_hbm.at[c, pl.ds(s * ROWS_PER_SUB, ROWS_PER_SUB)])

    return k(dst3)


def _sc_propagate(src2, dst2, v_pad):
    @functools.partial(
        pl.kernel,
        out_type=jax.ShapeDtypeStruct((NC, N_PAD, D), jnp.float32),
        mesh=_MESH,
        scratch_types=[
            pltpu.VMEM_SHARED((N_PAD, D), jnp.float32),
            pltpu.VMEM((N0C, CHUNK), jnp.int32),
            pltpu.VMEM((N0C, CHUNK), jnp.int32),
            pltpu.VMEM((ZR, D), jnp.float32),
            pltpu.VMEM((CHUNK, D), jnp.float32),
            pltpu.SemaphoreType.DMA,
        ],
    )
    def k(src2_hbm, dst2_hbm, v_hbm, part_hbm, acc, src_v, dst_v, zeros_v,
          rows_v, sem_g):
        c = lax.axis_index("c")
        s = lax.axis_index("s")
        cnt = jnp.where(c == 0, N0C, N1C)
        base = jnp.where(c == 0, s * N0C, NS * N0C + s * N1C)

        def fill_zeros(i, _):
            zeros_v[i // 8, pl.ds((i % 8) * 16, 16)] = jnp.zeros(
                (16,), jnp.float32)
            return 0

        lax.fori_loop(0, ZR * D // 16, fill_zeros, 0)

        def zero_stripe(i, _):
            pltpu.sync_copy(zeros_v,
                            acc.at[pl.ds(s * ROWS_PER_SUB + i * ZR, ZR)])
            return 0

        lax.fori_loop(0, ROWS_PER_SUB // ZR, zero_stripe, 0)
        plsc.subcore_barrier()

        pltpu.sync_copy(src2_hbm.at[pl.ds(base, N0C)], src_v)
        pltpu.sync_copy(dst2_hbm.at[pl.ds(base, N0C)], dst_v)

        def body(j, _):
            pltpu.async_copy(v_hbm.at[src_v.at[j]], rows_v, sem_g).wait()
            pltpu.sync_copy(rows_v, acc.at[dst_v.at[j]], add=True)
            return 0

        lax.fori_loop(0, cnt, body, 0)
        plsc.subcore_barrier()
        pltpu.sync_copy(acc.at[pl.ds(s * ROWS_PER_SUB, ROWS_PER_SUB)],
                        part_hbm.at[c, pl.ds(s * ROWS_PER_SUB, ROWS_PER_SUB)])

    return k(src2, dst2, v_pad)


def _dinv_block(degT_ref, row0, rows, check_valid=True):
    deg = degT_ref[:, 0:1] + degT_ref[:, 1:2]
    dv = lax.rsqrt(deg)
    ok = deg > 0
    if check_valid:
        ridx = row0 + lax.broadcasted_iota(jnp.int32, (rows, 1), 0)
        ok = ok & (ridx < N_NODES)
    return jnp.where(ok, dv, 0.0)


def _tc_scale_x(degT, x_pad):
    BR = 256

    def body(degT_ref, x_ref, xs_ref):
        i = pl.program_id(0)
        dv = _dinv_block(degT_ref, i * BR, BR)
        xs_ref[...] = dv * x_ref[...]

    return pl.pallas_call(
        body,
        grid=(N_PAD // BR,),
        in_specs=[pl.BlockSpec((BR, 2), lambda i: (i, 0)),
                  pl.BlockSpec((BR, D), lambda i: (i, 0))],
        out_specs=pl.BlockSpec((BR, D), lambda i: (i, 0)),
        out_shape=jax.ShapeDtypeStruct((N_PAD, D), jnp.float32),
    )(degT, x_pad)


def _tc_mlp(degT, s1, W1, b1r, W2):
    BR = 256

    def body(degT_ref, s1_ref, W1_ref, b1_ref, W2_ref, gs_ref):
        i = pl.program_id(0)
        dv = _dinv_block(degT_ref, i * BR, BR)
        t = dv * (s1_ref[0] + s1_ref[1])
        h = jnp.dot(t, W1_ref[...], preferred_element_type=jnp.float32)
        h = jnp.maximum(h + b1_ref[...], 0.0)
        g = jnp.dot(h, W2_ref[...], preferred_element_type=jnp.float32)
        gs_ref[...] = dv * g

    d1 = W1.shape[1]
    return pl.pallas_call(
        body,
        grid=(N_PAD // BR,),
        in_specs=[pl.BlockSpec((BR, 2), lambda i: (i, 0)),
                  pl.BlockSpec((NC, BR, D), lambda i: (0, i, 0)),
                  pl.BlockSpec(W1.shape, lambda i: (0, 0)),
                  pl.BlockSpec((1, d1), lambda i: (0, 0)),
                  pl.BlockSpec(W2.shape, lambda i: (0, 0))],
        out_specs=pl.BlockSpec((BR, D), lambda i: (i, 0)),
        out_shape=jax.ShapeDtypeStruct((N_PAD, D), jnp.float32),
    )(degT, s1, W1, b1r, W2)


def _tc_final(degT, s2, b2r):
    BR = 2000

    def body(degT_ref, s2_ref, b2_ref, out_ref):
        dv = _dinv_block(degT_ref, 0, BR, check_valid=False)
        out_ref[...] = jnp.maximum(dv * (s2_ref[0] + s2_ref[1]) + b2_ref[...],
                                   0.0)

    return pl.pallas_call(
        body,
        grid=(N_NODES // BR,),
        in_specs=[pl.BlockSpec((BR, 2), lambda i: (i, 0)),
                  pl.BlockSpec((NC, BR, D), lambda i: (0, i, 0)),
                  pl.BlockSpec((1, D), lambda i: (0, 0))],
        out_specs=pl.BlockSpec((BR, D), lambda i: (i, 0)),
        out_shape=jax.ShapeDtypeStruct((N_NODES, D), jnp.float32),
    )(degT, s2, b2r)


def kernel(x, edge_index, W1, b1, W2, b2):
    loop = jnp.arange(N_NODES, dtype=jnp.int32)
    src = jnp.concatenate([edge_index[0].astype(jnp.int32), loop])
    dst = jnp.concatenate([edge_index[1].astype(jnp.int32), loop])
    e_tot = src.shape[0]
    rows_used = NS * (N0C + N1C)
    rows_alloc = rows_used + (N0C - N1C)
    assert e_tot <= rows_used * CHUNK
    src2 = (jnp.full((rows_alloc * CHUNK,), PAD_SRC, jnp.int32)
            .at[:e_tot].set(src).reshape(rows_alloc, CHUNK))
    dst2 = (jnp.full((rows_alloc * CHUNK,), PAD_DST, jnp.int32)
            .at[:e_tot].set(dst).reshape(rows_alloc, CHUNK))
    dst3 = dst2[:rows_used].reshape(NW, rows_used // NW, CHUNK)
    x_pad = jnp.zeros((N_PAD, D), jnp.float32).at[:N_NODES].set(
        x.astype(jnp.float32))

    degp = _sc_degree(dst3)
    degT = jnp.transpose(degp)
    xs = _tc_scale_x(degT, x_pad)
    s1 = _sc_propagate(src2, dst2, xs)
    gs = _tc_mlp(degT, s1, W1.astype(jnp.float32),
                 b1.reshape(1, -1).astype(jnp.float32),
                 W2.astype(jnp.float32))
    s2 = _sc_propagate(src2, dst2, gs)
    out = _tc_final(degT, s2, b2.reshape(1, -1).astype(jnp.float32))
    return out


# speedup vs baseline: 2.5406x; 2.5406x over previous
import functools

import jax
import jax.numpy as jnp
from jax import lax
from jax.experimental import pallas as pl
from jax.experimental.pallas import tpu as pltpu
from jax.experimental.pallas import tpu_sc as plsc

N_NODES = 10000
D = 128
N_PAD = 10240
NC, NS = 2, 16
NW = NC * NS
CHUNK = 128
ROWS_PER_SUB = N_PAD // NS
ZR = 64
PAD_SRC = N_NODES
PAD_DST = N_NODES + 8

_MESH = plsc.VectorSubcoreMesh(core_axis_name="c", subcore_axis_name="s")


def _sc_degree(dst3):
    n_chunks = dst3.shape[1]

    @functools.partial(
        pl.kernel,
        out_type=jax.ShapeDtypeStruct((NC, N_PAD), jnp.float32),
        mesh=_MESH,
        compiler_params=pltpu.CompilerParams(needs_layout_passes=False),
        scratch_types=[
            pltpu.VMEM_SHARED((NS, N_PAD), jnp.float32),
            pltpu.VMEM((n_chunks, CHUNK), jnp.int32),
            pltpu.VMEM((N_PAD,), jnp.float32),
            pltpu.VMEM((NS, ROWS_PER_SUB), jnp.float32),
        ],
    )
    def k(dst3_hbm, degp_hbm, slab, idx_v, hist_v, win_v):
        c = lax.axis_index("c")
        s = lax.axis_index("s")
        wid = c * NS + s

        def fill_zeros(i, _):
            hist_v[pl.ds(i * 16, 16)] = jnp.zeros((16,), jnp.float32)
            return 0

        lax.fori_loop(0, N_PAD // 16, fill_zeros, 0)
        pltpu.sync_copy(dst3_hbm.at[wid], idx_v)
        ones16 = jnp.full((16,), 1.0, jnp.float32)

        def body(j, _):
            def inner(t, _):
                idx = idx_v[j, pl.ds(t * 16, 16)]
                plsc.addupdate_scatter(hist_v, [idx], ones16)
                return 0

            lax.fori_loop(0, CHUNK // 16, inner, 0)
            return 0

        lax.fori_loop(0, n_chunks, body, 0)
        pltpu.sync_copy(hist_v, slab.at[s])
        plsc.subcore_barrier()
        pltpu.sync_copy(slab.at[:, pl.ds(s * ROWS_PER_SUB, ROWS_PER_SUB)],
                        win_v)

        def red(j, _):
            acc = jnp.zeros((16,), jnp.float32)

            def radd(i, a):
                return a + win_v[i, pl.ds(j * 16, 16)]

            acc = lax.fori_loop(0, NS, radd, acc)
            hist_v[pl.ds(j * 16, 16)] = acc
            return 0

        lax.fori_loop(0, ROWS_PER_SUB // 16, red, 0)
        pltpu.sync_copy(hist_v.at[pl.ds(0, ROWS_PER_SUB)],
                        degp_hbm.at[c, pl.ds(s * ROWS_PER_SUB, ROWS_PER_SUB)])

    return k(dst3)


def _sc_propagate(src3, dst3, v_pad):
    n_chunks = src3.shape[1]

    @functools.partial(
        pl.kernel,
        out_type=jax.ShapeDtypeStruct((NC, N_PAD, D), jnp.float32),
        mesh=_MESH,
        scratch_types=[
            pltpu.VMEM_SHARED((N_PAD, D), jnp.float32),
            pltpu.VMEM((n_chunks, CHUNK), jnp.int32),
            pltpu.VMEM((n_chunks, CHUNK), jnp.int32),
            pltpu.VMEM((ZR, D), jnp.float32),
            pltpu.VMEM((CHUNK, D), jnp.float32),
            pltpu.SemaphoreType.DMA,
        ],
    )
    def k(src3_hbm, dst3_hbm, v_hbm, part_hbm, acc, src_v, dst_v, zeros_v,
          rows_v, sem_g):
        c = lax.axis_index("c")
        s = lax.axis_index("s")
        wid = c * NS + s

        def fill_zeros(i, _):
            zeros_v[i // 8, pl.ds((i % 8) * 16, 16)] = jnp.zeros(
                (16,), jnp.float32)
            return 0

        lax.fori_loop(0, ZR * D // 16, fill_zeros, 0)

        def zero_stripe(i, _):
            pltpu.sync_copy(zeros_v,
                            acc.at[pl.ds(s * ROWS_PER_SUB + i * ZR, ZR)])
            return 0

        lax.fori_loop(0, ROWS_PER_SUB // ZR, zero_stripe, 0)
        plsc.subcore_barrier()

        pltpu.sync_copy(src3_hbm.at[wid], src_v)
        pltpu.sync_copy(dst3_hbm.at[wid], dst_v)

        def body(j, _):
            pltpu.async_copy(v_hbm.at[src_v.at[j]], rows_v, sem_g).wait()
            pltpu.sync_copy(rows_v, acc.at[dst_v.at[j]], add=True)
            return 0

        lax.fori_loop(0, n_chunks, body, 0)
        plsc.subcore_barrier()
        pltpu.sync_copy(acc.at[pl.ds(s * ROWS_PER_SUB, ROWS_PER_SUB)],
                        part_hbm.at[c, pl.ds(s * ROWS_PER_SUB, ROWS_PER_SUB)])

    return k(src3, dst3, v_pad)


def _dinv_block(degT_ref, row0, rows, check_valid=True):
    deg = degT_ref[:, 0:1] + degT_ref[:, 1:2]
    dv = lax.rsqrt(deg)
    ok = deg > 0
    if check_valid:
        ridx = row0 + lax.broadcasted_iota(jnp.int32, (rows, 1), 0)
        ok = ok & (ridx < N_NODES)
    return jnp.where(ok, dv, 0.0)


def _tc_scale_x(degT, x_pad):
    BR = 256

    def body(degT_ref, x_ref, xs_ref):
        i = pl.program_id(0)
        dv = _dinv_block(degT_ref, i * BR, BR)
        xs_ref[...] = dv * x_ref[...]

    return pl.pallas_call(
        body,
        grid=(N_PAD // BR,),
        in_specs=[pl.BlockSpec((BR, 2), lambda i: (i, 0)),
                  pl.BlockSpec((BR, D), lambda i: (i, 0))],
        out_specs=pl.BlockSpec((BR, D), lambda i: (i, 0)),
        out_shape=jax.ShapeDtypeStruct((N_PAD, D), jnp.float32),
    )(degT, x_pad)


def _tc_mlp(degT, s1, W1, b1r, W2):
    BR = 256

    def body(degT_ref, s1_ref, W1_ref, b1_ref, W2_ref, gs_ref):
        i = pl.program_id(0)
        dv = _dinv_block(degT_ref, i * BR, BR)
        t = dv * (s1_ref[0] + s1_ref[1])
        h = jnp.dot(t, W1_ref[...], preferred_element_type=jnp.float32)
        h = jnp.maximum(h + b1_ref[...], 0.0)
        g = jnp.dot(h, W2_ref[...], preferred_element_type=jnp.float32)
        gs_ref[...] = dv * g

    d1 = W1.shape[1]
    return pl.pallas_call(
        body,
        grid=(N_PAD // BR,),
        in_specs=[pl.BlockSpec((BR, 2), lambda i: (i, 0)),
                  pl.BlockSpec((NC, BR, D), lambda i: (0, i, 0)),
                  pl.BlockSpec(W1.shape, lambda i: (0, 0)),
                  pl.BlockSpec((1, d1), lambda i: (0, 0)),
                  pl.BlockSpec(W2.shape, lambda i: (0, 0))],
        out_specs=pl.BlockSpec((BR, D), lambda i: (i, 0)),
        out_shape=jax.ShapeDtypeStruct((N_PAD, D), jnp.float32),
    )(degT, s1, W1, b1r, W2)


def _tc_final(degT, s2, b2r):
    BR = 2000

    def body(degT_ref, s2_ref, b2_ref, out_ref):
        dv = _dinv_block(degT_ref, 0, BR, check_valid=False)
        out_ref[...] = jnp.maximum(dv * (s2_ref[0] + s2_ref[1]) + b2_ref[...],
                                   0.0)

    return pl.pallas_call(
        body,
        grid=(N_NODES // BR,),
        in_specs=[pl.BlockSpec((BR, 2), lambda i: (i, 0)),
                  pl.BlockSpec((NC, BR, D), lambda i: (0, i, 0)),
                  pl.BlockSpec((1, D), lambda i: (0, 0))],
        out_specs=pl.BlockSpec((BR, D), lambda i: (i, 0)),
        out_shape=jax.ShapeDtypeStruct((N_NODES, D), jnp.float32),
    )(degT, s2, b2r)


def kernel(x, edge_index, W1, b1, W2, b2):
    loop = jnp.arange(N_NODES, dtype=jnp.int32)
    src = jnp.concatenate([edge_index[0].astype(jnp.int32), loop])
    dst = jnp.concatenate([edge_index[1].astype(jnp.int32), loop])
    e_tot = src.shape[0]
    per = NW * CHUNK
    e_pad = per * ((e_tot + per - 1) // per)
    src3 = (jnp.full((e_pad,), PAD_SRC, jnp.int32).at[:e_tot].set(src)
            .reshape(NW, -1, CHUNK))
    dst3 = (jnp.full((e_pad,), PAD_DST, jnp.int32).at[:e_tot].set(dst)
            .reshape(NW, -1, CHUNK))
    x_pad = jnp.zeros((N_PAD, D), jnp.float32).at[:N_NODES].set(
        x.astype(jnp.float32))

    degp = _sc_degree(dst3)
    degT = jnp.transpose(degp)
    xs = _tc_scale_x(degT, x_pad)
    s1 = _sc_propagate(src3, dst3, xs)
    gs = _tc_mlp(degT, s1, W1.astype(jnp.float32),
                 b1.reshape(1, -1).astype(jnp.float32),
                 W2.astype(jnp.float32))
    s2 = _sc_propagate(src3, dst3, gs)
    out = _tc_final(degT, s2, b2.reshape(1, -1).astype(jnp.float32))
    return out
